# baseline (speedup 1.0000x reference)
import functools

import jax
import jax.numpy as jnp
from jax import lax
from jax.experimental import pallas as pl
from jax.experimental.pallas import tpu as pltpu
from jax.experimental.pallas import tpu_sc as plsc

EMBED_DIM = 512
HIDDEN_DIM = 1024
BATCH = 16384

_NC = 2
_NS = 16
_NW = _NC * _NS
_CH = 64
_SLAB_SIZES = (6144, 4096, 4096, 2048)


def _make_gather_body(bpw, nchunk):
    def body(entity_hbm, rel_hbm, idx_hbm,
             out_s, out_r, out_o,
             *scratch):
        idx_refs = scratch[:3 * nchunk]
        (fbuf0, fbuf1, fbuf2, isem,
         gsem0, gsem1, gsem2, wsem0, wsem1, wsem2) = scratch[3 * nchunk:]
        wid = lax.axis_index("s") * _NC + lax.axis_index("c")
        base = wid * bpw

        icps = [pltpu.async_copy(idx_hbm.at[wid * 3 * nchunk + j],
                                 idx_refs[j], isem)
                for j in range(3 * nchunk)]
        for cp in icps:
            cp.wait()

        tables = (entity_hbm, rel_hbm, entity_hbm)
        outs = (out_s, out_r, out_o)
        fbufs = (fbuf0, fbuf1, fbuf2)
        gsems = (gsem0, gsem1, gsem2)
        wsems = (wsem0, wsem1, wsem2)
        nbuf = 3
        chunks = [(t, k) for t in range(3) for k in range(nchunk)]
        total = len(chunks)

        pend_g = [None] * nbuf
        pend_w = [None] * nbuf
        for c in range(total + nbuf - 1):
            if c < total:
                slot = c % nbuf
                if pend_w[slot] is not None:
                    pend_w[slot].wait()
                t, k = chunks[c]
                pend_g[slot] = pltpu.async_copy(
                    tables[t].at[idx_refs[t * nchunk + k]],
                    fbufs[slot], gsems[slot])
            d = c - (nbuf - 1)
            if 0 <= d < total:
                ds_ = d % nbuf
                td, kd = chunks[d]
                pend_g[ds_].wait()
                pend_w[ds_] = pltpu.async_copy(
                    fbufs[ds_], outs[td].at[pl.ds(base + kd * _CH, _CH)],
                    wsems[ds_])
        for p in pend_w:
            if p is not None:
                p.wait()

    return body


@functools.cache
def _sc_gather(sb):
    bpw = sb // _NW
    nchunk = bpw // _CH
    return functools.partial(
        pl.kernel,
        out_type=[jax.ShapeDtypeStruct((sb, EMBED_DIM), jnp.float32)] * 3,
        mesh=plsc.VectorSubcoreMesh(core_axis_name="c", subcore_axis_name="s",
                                    num_cores=_NC, num_subcores=_NS),
        scratch_types=(
            [pltpu.VMEM((_CH,), jnp.int32)] * (3 * nchunk) + [
                pltpu.VMEM((_CH, EMBED_DIM), jnp.float32),
                pltpu.VMEM((_CH, EMBED_DIM), jnp.float32),
                pltpu.VMEM((_CH, EMBED_DIM), jnp.float32),
                pltpu.SemaphoreType.DMA,
                pltpu.SemaphoreType.DMA,
                pltpu.SemaphoreType.DMA,
                pltpu.SemaphoreType.DMA,
                pltpu.SemaphoreType.DMA,
                pltpu.SemaphoreType.DMA,
                pltpu.SemaphoreType.DMA,
            ]),
    )(_make_gather_body(bpw, nchunk))


_BM = 512


def _mlp_body(hs_ref, hr_ref, ho_ref, w1s_ref, w1r_ref, w1o_ref,
              b1_ref, w2t_ref, b2_ref, out_ref):
    bf = jnp.bfloat16
    acc = jnp.dot(hs_ref[...].astype(bf), w1s_ref[...].astype(bf),
                  preferred_element_type=jnp.float32)
    acc += jnp.dot(hr_ref[...].astype(bf), w1r_ref[...].astype(bf),
                   preferred_element_type=jnp.float32)
    acc += jnp.dot(ho_ref[...].astype(bf), w1o_ref[...].astype(bf),
                   preferred_element_type=jnp.float32)
    hidden = jnp.maximum(acc + b1_ref[...], 0.0)
    out_ref[...] = jnp.sum(hidden * w2t_ref[...], axis=1) + b2_ref[0, 0]


def _mlp(sb, hs, hr, ho, fc1, b1, w2t, b2):
    grid = (sb // _BM,)
    wspec = lambda t: pl.BlockSpec((EMBED_DIM, HIDDEN_DIM),
                                   lambda i, _t=t: (_t, 0))
    return pl.pallas_call(
        _mlp_body,
        grid=grid,
        in_specs=[
            pl.BlockSpec((_BM, EMBED_DIM), lambda i: (i, 0)),
            pl.BlockSpec((_BM, EMBED_DIM), lambda i: (i, 0)),
            pl.BlockSpec((_BM, EMBED_DIM), lambda i: (i, 0)),
            wspec(0),
            wspec(1),
            wspec(2),
            pl.BlockSpec((1, HIDDEN_DIM), lambda i: (0, 0)),
            pl.BlockSpec((1, HIDDEN_DIM), lambda i: (0, 0)),
            pl.BlockSpec((1, 1), lambda i: (0, 0)),
        ],
        out_specs=pl.BlockSpec((_BM,), lambda i: (i,)),
        out_shape=jax.ShapeDtypeStruct((sb,), jnp.float32),
    )(hs, hr, ho, fc1, fc1, fc1, b1, w2t, b2)


def kernel(entity_emb, triples, rel_emb, fc1, fc1_bias, fc2, fc2_bias):
    idx = triples.astype(jnp.int32)
    b1 = fc1_bias.reshape(1, HIDDEN_DIM)
    w2t = fc2.reshape(1, HIDDEN_DIM)
    b2 = fc2_bias.reshape(1, 1)

    gathered = []
    lo = 0
    for sb in _SLAB_SIZES:
        bpw = sb // _NW
        idx_slab = lax.slice(idx, (lo, 0), (lo + sb, 3))
        idx_packed = idx_slab.reshape(_NW, bpw, 3).transpose(0, 2, 1)
        idx_packed = idx_packed.reshape(_NW * 3 * (bpw // _CH), _CH)
        gathered.append(_sc_gather(sb)(entity_emb, rel_emb, idx_packed))
        lo += sb
    outs = [_mlp(sb, hs, hr, ho, fc1, b1, w2t, b2)
            for sb, (hs, hr, ho) in zip(_SLAB_SIZES, gathered)]
    return jnp.concatenate(outs, axis=0)


# speedup vs baseline: 1.0019x; 1.0019x over previous
import functools

import jax
import jax.numpy as jnp
from jax import lax
from jax.experimental import pallas as pl
from jax.experimental.pallas import tpu as pltpu
from jax.experimental.pallas import tpu_sc as plsc

EMBED_DIM = 512
HIDDEN_DIM = 1024
BATCH = 16384

_NC = 2
_NS = 16
_NW = _NC * _NS
_CH = 64
_SLAB_SIZES = (6144, 4096, 4096, 2048)


def _make_gather_body(bpw, nchunk):
    def body(entity_hbm, rel_hbm, idx_hbm,
             out_s, out_r, out_o,
             *scratch):
        idx_refs = scratch[:3 * nchunk]
        (fbuf0, fbuf1, fbuf2, isem,
         gsem0, gsem1, gsem2, wsem0, wsem1, wsem2) = scratch[3 * nchunk:]
        wid = lax.axis_index("s") * _NC + lax.axis_index("c")
        base = wid * bpw
        fbufs = (fbuf0, fbuf1, fbuf2)

        icps = [pltpu.async_copy(idx_hbm.at[wid * 3 * nchunk + j],
                                 idx_refs[j], isem)
                for j in range(3 * nchunk)]
        for cp in icps:
            cp.wait()

        tables = (entity_hbm, rel_hbm, entity_hbm)
        outs = (out_s, out_r, out_o)
        gsems = (gsem0, gsem1, gsem2)
        wsems = (wsem0, wsem1, wsem2)
        nbuf = 3
        chunks = [(t, k) for t in range(3) for k in range(nchunk)]
        total = len(chunks)

        pend_g = [None] * nbuf
        pend_w = [None] * nbuf
        for c in range(total + nbuf - 1):
            if c < total:
                slot = c % nbuf
                if pend_w[slot] is not None:
                    pend_w[slot].wait()
                t, k = chunks[c]
                pend_g[slot] = pltpu.async_copy(
                    tables[t].at[idx_refs[t * nchunk + k]],
                    fbufs[slot], gsems[slot])
            d = c - (nbuf - 1)
            if 0 <= d < total:
                ds_ = d % nbuf
                td, kd = chunks[d]
                pend_g[ds_].wait()
                pend_w[ds_] = pltpu.async_copy(
                    fbufs[ds_], outs[td].at[pl.ds(base + kd * _CH, _CH)],
                    wsems[ds_])
        for p in pend_w:
            if p is not None:
                p.wait()

    return body


@functools.cache
def _sc_gather(sb):
    bpw = sb // _NW
    nchunk = bpw // _CH
    return functools.partial(
        pl.kernel,
        out_type=[jax.ShapeDtypeStruct((sb, EMBED_DIM), jnp.float32)] * 3,
        mesh=plsc.VectorSubcoreMesh(core_axis_name="c", subcore_axis_name="s",
                                    num_cores=_NC, num_subcores=_NS),
        scratch_types=(
            [pltpu.VMEM((_CH,), jnp.int32)] * (3 * nchunk) + [
                pltpu.VMEM((_CH, EMBED_DIM), jnp.float32),
                pltpu.VMEM((_CH, EMBED_DIM), jnp.float32),
                pltpu.VMEM((_CH, EMBED_DIM), jnp.float32),
                pltpu.SemaphoreType.DMA,
                pltpu.SemaphoreType.DMA,
                pltpu.SemaphoreType.DMA,
                pltpu.SemaphoreType.DMA,
                pltpu.SemaphoreType.DMA,
                pltpu.SemaphoreType.DMA,
                pltpu.SemaphoreType.DMA,
            ]),
    )(_make_gather_body(bpw, nchunk))


_BM = 512


def _mlp_body(hs_ref, hr_ref, ho_ref, w1s_ref, w1r_ref, w1o_ref,
              b1_ref, w2t_ref, b2_ref, out_ref):
    bf = jnp.bfloat16
    acc = jnp.dot(hs_ref[...].astype(bf), w1s_ref[...].astype(bf),
                  preferred_element_type=jnp.float32)
    acc += jnp.dot(hr_ref[...].astype(bf), w1r_ref[...].astype(bf),
                   preferred_element_type=jnp.float32)
    acc += jnp.dot(ho_ref[...].astype(bf), w1o_ref[...].astype(bf),
                   preferred_element_type=jnp.float32)
    hidden = jnp.maximum(acc + b1_ref[...], 0.0)
    out_ref[...] = jnp.sum(hidden * w2t_ref[...], axis=1) + b2_ref[0, 0]


def _mlp(sb, hs, hr, ho, fc1, b1, w2t, b2):
    grid = (sb // _BM,)
    wspec = lambda t: pl.BlockSpec((EMBED_DIM, HIDDEN_DIM),
                                   lambda i, _t=t: (_t, 0))
    return pl.pallas_call(
        _mlp_body,
        grid=grid,
        in_specs=[
            pl.BlockSpec((_BM, EMBED_DIM), lambda i: (i, 0)),
            pl.BlockSpec((_BM, EMBED_DIM), lambda i: (i, 0)),
            pl.BlockSpec((_BM, EMBED_DIM), lambda i: (i, 0)),
            wspec(0),
            wspec(1),
            wspec(2),
            pl.BlockSpec((1, HIDDEN_DIM), lambda i: (0, 0)),
            pl.BlockSpec((1, HIDDEN_DIM), lambda i: (0, 0)),
            pl.BlockSpec((1, 1), lambda i: (0, 0)),
        ],
        out_specs=pl.BlockSpec((_BM,), lambda i: (i,)),
        out_shape=jax.ShapeDtypeStruct((sb,), jnp.float32),
    )(hs, hr, ho, fc1, fc1, fc1, b1, w2t, b2)


def kernel(entity_emb, triples, rel_emb, fc1, fc1_bias, fc2, fc2_bias):
    idx = triples.astype(jnp.int32)
    b1 = fc1_bias.reshape(1, HIDDEN_DIM)
    w2t = fc2.reshape(1, HIDDEN_DIM)
    b2 = fc2_bias.reshape(1, 1)

    gathered = []
    lo = 0
    for sb in _SLAB_SIZES:
        bpw = sb // _NW
        idx_slab = lax.slice(idx, (lo, 0), (lo + sb, 3))
        idx_packed = idx_slab.reshape(_NW, bpw, 3).transpose(0, 2, 1)
        idx_packed = idx_packed.reshape(_NW * 3 * (bpw // _CH), _CH)
        gathered.append(_sc_gather(sb)(entity_emb, rel_emb, idx_packed))
        lo += sb
    outs = [_mlp(sb, hs, hr, ho, fc1, b1, w2t, b2)
            for sb, (hs, hr, ho) in zip(_SLAB_SIZES, gathered)]
    return jnp.concatenate(outs, axis=0)
